# baseline (speedup 1.0000x reference)
import functools

import jax
import jax.numpy as jnp
from jax import lax
from jax.experimental import pallas as pl
from jax.experimental.pallas import tpu as pltpu
from jax.experimental.pallas import tpu_sc as plsc

N = 10000
E = 320000
D = 128

NC = 2
NS = 16
NW = NC * NS
CHUNK = 128
NBUF = 2
NCH = 80
NG = NCH // NBUF
EPW = NCH * CHUNK
EPAD = NW * EPW

NP = 10240
RPS = NP // NS
PAD_DST = N + 8

DW = 16

_sc_mesh = plsc.VectorSubcoreMesh(
    core_axis_name="c", subcore_axis_name="s", num_cores=NC, num_subcores=NS)


def _worker_id():
    cid = lax.axis_index("c")
    sid = lax.axis_index("s")
    return cid, sid, sid * NC + cid


@functools.partial(
    pl.kernel,
    out_type=[jax.ShapeDtypeStruct((NP, DW), jnp.float32),
              jax.ShapeDtypeStruct((NP, DW), jnp.float32)],
    mesh=_sc_mesh,
    scratch_types=[
        pltpu.VMEM_SHARED((NP, DW), jnp.float32),
        pltpu.VMEM((CHUNK,), jnp.int32),
        pltpu.VMEM((CHUNK,), jnp.int32),
        pltpu.VMEM((CHUNK, DW), jnp.float32),
        pltpu.VMEM((64, DW), jnp.float32),
        pltpu.SemaphoreType.DMA,
        pltpu.SemaphoreType.DMA,
        pltpu.SemaphoreType.DMA,
        pltpu.SemaphoreType.DMA,
    ],
)
def _sc_degree(dstp_hbm, deg0_hbm, deg1_hbm, acc_sh, dst_v0, dst_v1, ones_v,
               zer_v, si0, si1, ss0, ss1):
    dst_v = (dst_v0, dst_v1)
    sem_i = (si0, si1)
    sem_s = (ss0, ss1)
    cid, sid, wid = _worker_id()
    one = jnp.ones((16,), jnp.float32)
    zero = jnp.zeros((16,), jnp.float32)
    for i in range(CHUNK):
        ones_v[i] = one
    for i in range(64):
        zer_v[i] = zero
    for k in range(RPS // 64):
        pltpu.sync_copy(zer_v, acc_sh.at[pl.ds(sid * RPS + k * 64, 64)])
    plsc.subcore_barrier()

    for b in range(2):
        pltpu.async_copy(dstp_hbm.at[wid, b], dst_v[b], sem_i[b])

    def body(g, carry):
        for b in range(2):
            pltpu.make_async_copy(dstp_hbm.at[wid, 0], dst_v[b],
                                  sem_i[b]).wait()
            pltpu.async_copy(ones_v, acc_sh.at[dst_v[b]], sem_s[b], add=True)

        @pl.when(g < NCH // 2 - 1)
        def _():
            for b in range(2):
                pltpu.make_async_copy(ones_v, acc_sh.at[dst_v[b]],
                                      sem_s[b]).wait()
                pltpu.async_copy(dstp_hbm.at[wid, 2 * g + 2 + b], dst_v[b],
                                 sem_i[b])

        return carry

    lax.fori_loop(0, NCH // 2, body, 0)
    for b in range(2):
        pltpu.make_async_copy(ones_v, acc_sh.at[dst_v[b]], sem_s[b]).wait()
    plsc.subcore_barrier()

    @pl.when(cid == 0)
    def _():
        pltpu.sync_copy(acc_sh.at[pl.ds(sid * RPS, RPS)],
                        deg0_hbm.at[pl.ds(sid * RPS, RPS)])

    @pl.when(cid == 1)
    def _():
        pltpu.sync_copy(acc_sh.at[pl.ds(sid * RPS, RPS)],
                        deg1_hbm.at[pl.ds(sid * RPS, RPS)])


@functools.partial(
    pl.kernel,
    out_type=[jax.ShapeDtypeStruct((NP, D), jnp.float32),
              jax.ShapeDtypeStruct((NP, D), jnp.float32)],
    mesh=_sc_mesh,
    scratch_types=[
        pltpu.VMEM_SHARED((NP, D), jnp.float32),
        pltpu.VMEM((NCH, CHUNK), jnp.int32),
    ] + [pltpu.VMEM((CHUNK, D), jnp.float32)] * NBUF
      + [pltpu.VMEM((CHUNK,), jnp.int32)] * NBUF
      + [pltpu.SemaphoreType.DMA] * (3 * NBUF),
)
def _sc_aggregate(hs_hbm, srcp_hbm, dstp_hbm, p0_hbm, p1_hbm,
                  acc_sh, src_all, *bufs_and_sems):
    rows = bufs_and_sems[:NBUF]
    dst_v = bufs_and_sems[NBUF:2 * NBUF]
    sem_g = bufs_and_sems[2 * NBUF:3 * NBUF]
    sem_i = bufs_and_sems[3 * NBUF:4 * NBUF]
    sem_s = bufs_and_sems[4 * NBUF:]
    cid, sid, wid = _worker_id()

    pltpu.sync_copy(srcp_hbm.at[wid], src_all)
    pltpu.sync_copy(hs_hbm.at[pl.ds(sid * RPS, RPS)],
                    acc_sh.at[pl.ds(sid * RPS, RPS)])
    plsc.subcore_barrier()

    for b in range(NBUF):
        pltpu.async_copy(hs_hbm.at[src_all.at[b]], rows[b], sem_g[b])
        pltpu.async_copy(dstp_hbm.at[wid, b], dst_v[b], sem_i[b])

    def body(g, carry):
        for b in range(NBUF):
            pltpu.make_async_copy(hs_hbm.at[src_all.at[0]], rows[b],
                                  sem_g[b]).wait()
            pltpu.make_async_copy(dstp_hbm.at[wid, 0], dst_v[b],
                                  sem_i[b]).wait()
            pltpu.async_copy(rows[b], acc_sh.at[dst_v[b]], sem_s[b],
                             add=True)
            pltpu.make_async_copy(rows[b], acc_sh.at[dst_v[b]],
                                  sem_s[b]).wait()

            @pl.when(g < NG - 1)
            def _():
                pltpu.async_copy(hs_hbm.at[src_all.at[NBUF * g + NBUF + b]],
                                 rows[b], sem_g[b])
                pltpu.async_copy(dstp_hbm.at[wid, NBUF * g + NBUF + b],
                                 dst_v[b], sem_i[b])

        return carry

    lax.fori_loop(0, NG, body, 0)
    plsc.subcore_barrier()

    @pl.when(cid == 0)
    def _():
        pltpu.sync_copy(acc_sh.at[pl.ds(sid * RPS, RPS)],
                        p0_hbm.at[pl.ds(sid * RPS, RPS)])

    @pl.when(cid == 1)
    def _():
        pltpu.sync_copy(acc_sh.at[pl.ds(sid * RPS, RPS)],
                        p1_hbm.at[pl.ds(sid * RPS, RPS)])


BR = 640


def _tc1_body(x_ref, w_ref, d0_ref, d1_ref, hs_ref, dinv_ref):
    cnt = d0_ref[:, 0:1] + d1_ref[:, 0:1] + 1.0
    dinv = lax.rsqrt(cnt)
    h = jnp.dot(x_ref[...], w_ref[...], preferred_element_type=jnp.float32)
    hs_ref[...] = dinv * h
    dinv_ref[...] = jnp.broadcast_to(dinv, (BR, DW))


def _tc2_body(p0_ref, p1_ref, hs_ref, dinv_ref, w_ref, b_ref, out_ref):
    dinv = dinv_ref[:, 0:1]
    agg = p0_ref[...] + p1_ref[...] - hs_ref[...]
    a = jnp.maximum(dinv * agg + b_ref[...], 0.0)
    out_ref[...] = dinv * jnp.dot(a, w_ref[...],
                                  preferred_element_type=jnp.float32)


def _tc3_body(p0_ref, p1_ref, hs_ref, dinv_ref, b2_ref,
              wf1_ref, bf1_ref, wf2_ref, bf2_ref, out_ref):
    dinv = dinv_ref[:, 0:1]
    agg = p0_ref[...] + p1_ref[...] - hs_ref[...]
    a = jnp.maximum(dinv * agg + b2_ref[...], 0.0)
    f = jnp.maximum(jnp.dot(a, wf1_ref[...],
                            preferred_element_type=jnp.float32) + bf1_ref[...],
                    0.0)
    logits = jnp.dot(f, wf2_ref[...],
                     preferred_element_type=jnp.float32) + bf2_ref[...]
    m = jnp.max(logits, axis=1, keepdims=True)
    e = jnp.exp(logits - m)
    out_ref[...] = e / jnp.sum(e, axis=1, keepdims=True)


def _row_spec(width):
    return pl.BlockSpec((BR, width), lambda i: (i, 0))


def _full_spec(shape):
    return pl.BlockSpec(shape, lambda i: (0,) * len(shape))


def _tc1(x, w1, deg0, deg1):
    return pl.pallas_call(
        _tc1_body,
        grid=(NP // BR,),
        in_specs=[_row_spec(D), _full_spec((D, D)), _row_spec(DW),
                  _row_spec(DW)],
        out_specs=[_row_spec(D), _row_spec(DW)],
        out_shape=[jax.ShapeDtypeStruct((NP, D), jnp.float32),
                   jax.ShapeDtypeStruct((NP, DW), jnp.float32)],
    )(x, w1, deg0, deg1)


def _tc2(p0, p1, hs, dinv, w2, b1):
    return pl.pallas_call(
        _tc2_body,
        grid=(NP // BR,),
        in_specs=[_row_spec(D), _row_spec(D), _row_spec(D), _row_spec(DW),
                  _full_spec((D, D)), _full_spec((1, D))],
        out_specs=_row_spec(D),
        out_shape=jax.ShapeDtypeStruct((NP, D), jnp.float32),
    )(p0, p1, hs, dinv, w2, b1)


def _tc3(p0, p1, hs, dinv, b2, wf1, bf1, wf2, bf2):
    return pl.pallas_call(
        _tc3_body,
        grid=(NP // BR,),
        in_specs=[_row_spec(D), _row_spec(D), _row_spec(D), _row_spec(DW),
                  _full_spec((1, D)), _full_spec((D, D)), _full_spec((1, D)),
                  _full_spec((D, D)), _full_spec((1, D))],
        out_specs=_row_spec(D),
        out_shape=jax.ShapeDtypeStruct((N, D), jnp.float32),
    )(p0, p1, hs, dinv, b2, wf1, bf1, wf2, bf2)


@jax.jit
def kernel(x, edge_index, W1, b1, W2, b2, Wf1, bf1, Wf2, bf2):
    src = edge_index[0]
    dst = edge_index[1]
    npad = EPAD - E
    srcp = jnp.concatenate(
        [src, jnp.zeros((npad,), jnp.int32)]).reshape(NW, NCH, CHUNK)
    dstp = jnp.concatenate(
        [dst, jnp.full((npad,), PAD_DST, jnp.int32)]).reshape(NW, NCH, CHUNK)

    deg0, deg1 = _sc_degree(dstp)

    hs1, dinv = _tc1(x, W1, deg0, deg1)
    p0, p1 = _sc_aggregate(hs1, srcp, dstp)
    hs2 = _tc2(p0, p1, hs1, dinv, W2, b1.reshape(1, D))
    q0, q1 = _sc_aggregate(hs2, srcp, dstp)
    out = _tc3(q0, q1, hs2, dinv, b2.reshape(1, D),
               Wf1, bf1.reshape(1, D), Wf2, bf2.reshape(1, D))
    return out


# speedup vs baseline: 1.2778x; 1.2778x over previous
import functools

import jax
import jax.numpy as jnp
from jax import lax
from jax.experimental import pallas as pl
from jax.experimental.pallas import tpu as pltpu
from jax.experimental.pallas import tpu_sc as plsc

N = 10000
E = 320000
D = 128

NC = 2
NS = 16
NW = NC * NS
CHUNK = 128
NBUF = 2
NCH = 80
NCHS = 160
NCH0 = 120
NCH1 = NCHS - NCH0
NG0 = NCH0 // NBUF
NG1 = NCH1 // NBUF
EPW = NCH * CHUNK
EPAD = NW * EPW

NP = 10240
RPS = NP // NS
PAD_DST = N + 8

DW = 16

_sc_mesh = plsc.VectorSubcoreMesh(
    core_axis_name="c", subcore_axis_name="s", num_cores=NC, num_subcores=NS)


def _worker_id():
    cid = lax.axis_index("c")
    sid = lax.axis_index("s")
    return cid, sid, sid * NC + cid


@functools.partial(
    pl.kernel,
    out_type=[jax.ShapeDtypeStruct((NP, DW), jnp.float32),
              jax.ShapeDtypeStruct((NP, DW), jnp.float32)],
    mesh=_sc_mesh,
    scratch_types=[
        pltpu.VMEM_SHARED((NP, DW), jnp.float32),
        pltpu.VMEM((CHUNK,), jnp.int32),
        pltpu.VMEM((CHUNK,), jnp.int32),
        pltpu.VMEM((CHUNK, DW), jnp.float32),
        pltpu.VMEM((64, DW), jnp.float32),
        pltpu.SemaphoreType.DMA,
        pltpu.SemaphoreType.DMA,
        pltpu.SemaphoreType.DMA,
        pltpu.SemaphoreType.DMA,
    ],
)
def _sc_degree(dstp_hbm, deg0_hbm, deg1_hbm, acc_sh, dst_v0, dst_v1, ones_v,
               zer_v, si0, si1, ss0, ss1):
    dst_v = (dst_v0, dst_v1)
    sem_i = (si0, si1)
    sem_s = (ss0, ss1)
    cid, sid, wid = _worker_id()
    one = jnp.ones((16,), jnp.float32)
    zero = jnp.zeros((16,), jnp.float32)
    for i in range(CHUNK):
        ones_v[i] = one
    for i in range(64):
        zer_v[i] = zero
    for k in range(RPS // 64):
        pltpu.sync_copy(zer_v, acc_sh.at[pl.ds(sid * RPS + k * 64, 64)])
    plsc.subcore_barrier()

    for b in range(2):
        pltpu.async_copy(dstp_hbm.at[wid, b], dst_v[b], sem_i[b])

    def body(g, carry):
        for b in range(2):
            pltpu.make_async_copy(dstp_hbm.at[wid, 0], dst_v[b],
                                  sem_i[b]).wait()
            pltpu.async_copy(ones_v, acc_sh.at[dst_v[b]], sem_s[b], add=True)

        @pl.when(g < NCH // 2 - 1)
        def _():
            for b in range(2):
                pltpu.make_async_copy(ones_v, acc_sh.at[dst_v[b]],
                                      sem_s[b]).wait()
                pltpu.async_copy(dstp_hbm.at[wid, 2 * g + 2 + b], dst_v[b],
                                 sem_i[b])

        return carry

    lax.fori_loop(0, NCH // 2, body, 0)
    for b in range(2):
        pltpu.make_async_copy(ones_v, acc_sh.at[dst_v[b]], sem_s[b]).wait()
    plsc.subcore_barrier()

    @pl.when(cid == 0)
    def _():
        pltpu.sync_copy(acc_sh.at[pl.ds(sid * RPS, RPS)],
                        deg0_hbm.at[pl.ds(sid * RPS, RPS)])

    @pl.when(cid == 1)
    def _():
        pltpu.sync_copy(acc_sh.at[pl.ds(sid * RPS, RPS)],
                        deg1_hbm.at[pl.ds(sid * RPS, RPS)])


@functools.partial(
    pl.kernel,
    out_type=[jax.ShapeDtypeStruct((NP, D), jnp.float32),
              jax.ShapeDtypeStruct((NP, D), jnp.float32)],
    mesh=_sc_mesh,
    scratch_types=[
        pltpu.VMEM_SHARED((NP, D), jnp.float32),
        pltpu.VMEM((NCH0, CHUNK), jnp.int32),
    ] + [pltpu.VMEM((CHUNK, D), jnp.float32)] * NBUF
      + [pltpu.VMEM((CHUNK,), jnp.int32)] * NBUF
      + [pltpu.SemaphoreType.DMA] * (3 * NBUF),
)
def _sc_aggregate(hs_hbm, srcp_hbm, dstp_hbm, p0_hbm, p1_hbm,
                  acc_sh, src_all, *bufs_and_sems):
    rows = bufs_and_sems[:NBUF]
    dst_v = bufs_and_sems[NBUF:2 * NBUF]
    sem_g = bufs_and_sems[2 * NBUF:3 * NBUF]
    sem_i = bufs_and_sems[3 * NBUF:4 * NBUF]
    sem_s = bufs_and_sems[4 * NBUF:]
    cid, sid, wid = _worker_id()
    off = cid * NCH0
    ng = jnp.where(cid == 0, NG0, NG1)

    @pl.when(cid == 0)
    def _():
        pltpu.sync_copy(srcp_hbm.at[sid, pl.ds(0, NCH0)], src_all)

    @pl.when(cid == 1)
    def _():
        pltpu.sync_copy(srcp_hbm.at[sid, pl.ds(NCH0, NCH1)],
                        src_all.at[pl.ds(0, NCH1)])

    pltpu.sync_copy(hs_hbm.at[pl.ds(sid * RPS, RPS)],
                    acc_sh.at[pl.ds(sid * RPS, RPS)])
    plsc.subcore_barrier()

    for b in range(NBUF):
        pltpu.async_copy(hs_hbm.at[src_all.at[b]], rows[b], sem_g[b])
        pltpu.async_copy(dstp_hbm.at[sid, off + b], dst_v[b], sem_i[b])

    def body(g, carry):
        for b in range(NBUF):
            pltpu.make_async_copy(hs_hbm.at[src_all.at[0]], rows[b],
                                  sem_g[b]).wait()
            pltpu.make_async_copy(dstp_hbm.at[sid, 0], dst_v[b],
                                  sem_i[b]).wait()
            pltpu.async_copy(rows[b], acc_sh.at[dst_v[b]], sem_s[b],
                             add=True)
            pltpu.make_async_copy(rows[b], acc_sh.at[dst_v[b]],
                                  sem_s[b]).wait()

            @pl.when(g < ng - 1)
            def _():
                pltpu.async_copy(hs_hbm.at[src_all.at[NBUF * g + NBUF + b]],
                                 rows[b], sem_g[b])
                pltpu.async_copy(dstp_hbm.at[sid, off + NBUF * g + NBUF + b],
                                 dst_v[b], sem_i[b])

        return carry

    lax.fori_loop(0, ng, body, 0)
    plsc.subcore_barrier()

    @pl.when(cid == 0)
    def _():
        pltpu.sync_copy(acc_sh.at[pl.ds(sid * RPS, RPS)],
                        p0_hbm.at[pl.ds(sid * RPS, RPS)])

    @pl.when(cid == 1)
    def _():
        pltpu.sync_copy(acc_sh.at[pl.ds(sid * RPS, RPS)],
                        p1_hbm.at[pl.ds(sid * RPS, RPS)])


BR = 640


def _tc1_body(x_ref, w_ref, d0_ref, d1_ref, hs_ref, dinv_ref):
    cnt = d0_ref[:, 0:1] + d1_ref[:, 0:1] + 1.0
    dinv = lax.rsqrt(cnt)
    h = jnp.dot(x_ref[...], w_ref[...], preferred_element_type=jnp.float32)
    hs_ref[...] = dinv * h
    dinv_ref[...] = jnp.broadcast_to(dinv, (BR, DW))


def _tc2_body(p0_ref, p1_ref, hs_ref, dinv_ref, w_ref, b_ref, out_ref):
    dinv = dinv_ref[:, 0:1]
    agg = p0_ref[...] + p1_ref[...] - hs_ref[...]
    a = jnp.maximum(dinv * agg + b_ref[...], 0.0)
    out_ref[...] = dinv * jnp.dot(a, w_ref[...],
                                  preferred_element_type=jnp.float32)


def _tc3_body(p0_ref, p1_ref, hs_ref, dinv_ref, b2_ref,
              wf1_ref, bf1_ref, wf2_ref, bf2_ref, out_ref):
    dinv = dinv_ref[:, 0:1]
    agg = p0_ref[...] + p1_ref[...] - hs_ref[...]
    a = jnp.maximum(dinv * agg + b2_ref[...], 0.0)
    f = jnp.maximum(jnp.dot(a, wf1_ref[...],
                            preferred_element_type=jnp.float32) + bf1_ref[...],
                    0.0)
    logits = jnp.dot(f, wf2_ref[...],
                     preferred_element_type=jnp.float32) + bf2_ref[...]
    m = jnp.max(logits, axis=1, keepdims=True)
    e = jnp.exp(logits - m)
    out_ref[...] = e / jnp.sum(e, axis=1, keepdims=True)


def _row_spec(width):
    return pl.BlockSpec((BR, width), lambda i: (i, 0))


def _full_spec(shape):
    return pl.BlockSpec(shape, lambda i: (0,) * len(shape))


def _tc1(x, w1, deg0, deg1):
    return pl.pallas_call(
        _tc1_body,
        grid=(NP // BR,),
        in_specs=[_row_spec(D), _full_spec((D, D)), _row_spec(DW),
                  _row_spec(DW)],
        out_specs=[_row_spec(D), _row_spec(DW)],
        out_shape=[jax.ShapeDtypeStruct((NP, D), jnp.float32),
                   jax.ShapeDtypeStruct((NP, DW), jnp.float32)],
    )(x, w1, deg0, deg1)


def _tc2(p0, p1, hs, dinv, w2, b1):
    return pl.pallas_call(
        _tc2_body,
        grid=(NP // BR,),
        in_specs=[_row_spec(D), _row_spec(D), _row_spec(D), _row_spec(DW),
                  _full_spec((D, D)), _full_spec((1, D))],
        out_specs=_row_spec(D),
        out_shape=jax.ShapeDtypeStruct((NP, D), jnp.float32),
    )(p0, p1, hs, dinv, w2, b1)


def _tc3(p0, p1, hs, dinv, b2, wf1, bf1, wf2, bf2):
    return pl.pallas_call(
        _tc3_body,
        grid=(NP // BR,),
        in_specs=[_row_spec(D), _row_spec(D), _row_spec(D), _row_spec(DW),
                  _full_spec((1, D)), _full_spec((D, D)), _full_spec((1, D)),
                  _full_spec((D, D)), _full_spec((1, D))],
        out_specs=_row_spec(D),
        out_shape=jax.ShapeDtypeStruct((N, D), jnp.float32),
    )(p0, p1, hs, dinv, b2, wf1, bf1, wf2, bf2)


@jax.jit
def kernel(x, edge_index, W1, b1, W2, b2, Wf1, bf1, Wf2, bf2):
    src = edge_index[0]
    dst = edge_index[1]
    npad = EPAD - E
    srcp = jnp.concatenate(
        [src, jnp.zeros((npad,), jnp.int32)]).reshape(NS, NCHS, CHUNK)
    dstp = jnp.concatenate(
        [dst, jnp.full((npad,), PAD_DST, jnp.int32)]).reshape(NS, NCHS, CHUNK)

    deg0, deg1 = _sc_degree(dstp.reshape(NW, NCH, CHUNK))

    hs1, dinv = _tc1(x, W1, deg0, deg1)
    p0, p1 = _sc_aggregate(hs1, srcp, dstp)
    hs2 = _tc2(p0, p1, hs1, dinv, W2, b1.reshape(1, D))
    q0, q1 = _sc_aggregate(hs2, srcp, dstp)
    out = _tc3(q0, q1, hs2, dinv, b2.reshape(1, D),
               Wf1, bf1.reshape(1, D), Wf2, bf2.reshape(1, D))
    return out


# speedup vs baseline: 1.3047x; 1.0210x over previous
import functools

import jax
import jax.numpy as jnp
from jax import lax
from jax.experimental import pallas as pl
from jax.experimental.pallas import tpu as pltpu
from jax.experimental.pallas import tpu_sc as plsc

N = 10000
E = 320000
D = 128

NC = 2
NS = 16
NW = NC * NS
CHUNK = 128
NBUF = 2
NCH = 80
NCHS = 160
NCH0 = 120
NCH1 = NCHS - NCH0
NG0 = NCH0 // NBUF
NG1 = NCH1 // NBUF
EPW = NCH * CHUNK
EPAD = NW * EPW

NP = 10240
RPS = NP // NS
PAD_DST = N + 8

DW = 16

_sc_mesh = plsc.VectorSubcoreMesh(
    core_axis_name="c", subcore_axis_name="s", num_cores=NC, num_subcores=NS)


def _worker_id():
    cid = lax.axis_index("c")
    sid = lax.axis_index("s")
    return cid, sid, sid * NC + cid


@functools.partial(
    pl.kernel,
    out_type=[jax.ShapeDtypeStruct((NP, DW), jnp.float32),
              jax.ShapeDtypeStruct((NP, DW), jnp.float32)],
    mesh=_sc_mesh,
    scratch_types=[
        pltpu.VMEM_SHARED((NP, DW), jnp.float32),
        pltpu.VMEM((CHUNK,), jnp.int32),
        pltpu.VMEM((CHUNK,), jnp.int32),
        pltpu.VMEM((CHUNK, DW), jnp.float32),
        pltpu.VMEM((64, DW), jnp.float32),
        pltpu.SemaphoreType.DMA,
        pltpu.SemaphoreType.DMA,
        pltpu.SemaphoreType.DMA,
        pltpu.SemaphoreType.DMA,
    ],
)
def _sc_degree(dstp_hbm, deg0_hbm, deg1_hbm, acc_sh, dst_v0, dst_v1, ones_v,
               zer_v, si0, si1, ss0, ss1):
    dst_v = (dst_v0, dst_v1)
    sem_i = (si0, si1)
    sem_s = (ss0, ss1)
    cid, sid, wid = _worker_id()
    one = jnp.ones((16,), jnp.float32)
    zero = jnp.zeros((16,), jnp.float32)
    for i in range(CHUNK):
        ones_v[i] = one
    for i in range(64):
        zer_v[i] = zero
    for k in range(RPS // 64):
        pltpu.sync_copy(zer_v, acc_sh.at[pl.ds(sid * RPS + k * 64, 64)])
    plsc.subcore_barrier()

    for b in range(2):
        pltpu.async_copy(dstp_hbm.at[wid, b], dst_v[b], sem_i[b])

    def body(g, carry):
        for b in range(2):
            pltpu.make_async_copy(dstp_hbm.at[wid, 0], dst_v[b],
                                  sem_i[b]).wait()
            pltpu.async_copy(ones_v, acc_sh.at[dst_v[b]], sem_s[b], add=True)

        @pl.when(g < NCH // 2 - 1)
        def _():
            for b in range(2):
                pltpu.make_async_copy(ones_v, acc_sh.at[dst_v[b]],
                                      sem_s[b]).wait()
                pltpu.async_copy(dstp_hbm.at[wid, 2 * g + 2 + b], dst_v[b],
                                 sem_i[b])

        return carry

    lax.fori_loop(0, NCH // 2, body, 0)
    for b in range(2):
        pltpu.make_async_copy(ones_v, acc_sh.at[dst_v[b]], sem_s[b]).wait()
    plsc.subcore_barrier()

    @pl.when(cid == 0)
    def _():
        pltpu.sync_copy(acc_sh.at[pl.ds(sid * RPS, RPS)],
                        deg0_hbm.at[pl.ds(sid * RPS, RPS)])

    @pl.when(cid == 1)
    def _():
        pltpu.sync_copy(acc_sh.at[pl.ds(sid * RPS, RPS)],
                        deg1_hbm.at[pl.ds(sid * RPS, RPS)])


@functools.partial(
    pl.kernel,
    out_type=[jax.ShapeDtypeStruct((NP, D), jnp.float32),
              jax.ShapeDtypeStruct((NP, D), jnp.float32)],
    mesh=_sc_mesh,
    scratch_types=[
        pltpu.VMEM_SHARED((NP, D), jnp.float32),
        pltpu.VMEM((NCH0, CHUNK), jnp.int32),
    ] + [pltpu.VMEM((CHUNK, D), jnp.float32)] * NBUF
      + [pltpu.VMEM((CHUNK,), jnp.int32)] * NBUF
      + [pltpu.SemaphoreType.DMA] * (3 * NBUF),
)
def _sc_aggregate(hs_hbm, srcp_hbm, dstp_hbm, p0_hbm, p1_hbm,
                  acc_sh, src_all, *bufs_and_sems):
    rows = bufs_and_sems[:NBUF]
    dst_v = bufs_and_sems[NBUF:2 * NBUF]
    sem_g = bufs_and_sems[2 * NBUF:3 * NBUF]
    sem_i = bufs_and_sems[3 * NBUF:4 * NBUF]
    sem_s = bufs_and_sems[4 * NBUF:]
    cid, sid, wid = _worker_id()
    off = cid * NCH0
    ng = jnp.where(cid == 0, NG0, NG1)

    pltpu.sync_copy(srcp_hbm.at[sid * NC + cid], src_all)
    pltpu.sync_copy(hs_hbm.at[pl.ds(sid * RPS, RPS)],
                    acc_sh.at[pl.ds(sid * RPS, RPS)])
    plsc.subcore_barrier()

    for b in range(NBUF):
        pltpu.async_copy(hs_hbm.at[src_all.at[b]], rows[b], sem_g[b])
        pltpu.async_copy(dstp_hbm.at[sid, off + b], dst_v[b], sem_i[b])

    def body(g, carry):
        for b in range(NBUF):
            pltpu.make_async_copy(hs_hbm.at[src_all.at[0]], rows[b],
                                  sem_g[b]).wait()
            pltpu.make_async_copy(dstp_hbm.at[sid, 0], dst_v[b],
                                  sem_i[b]).wait()
            pltpu.async_copy(rows[b], acc_sh.at[dst_v[b]], sem_s[b],
                             add=True)
            pltpu.make_async_copy(rows[b], acc_sh.at[dst_v[b]],
                                  sem_s[b]).wait()

            @pl.when(g < ng - 1)
            def _():
                pltpu.async_copy(hs_hbm.at[src_all.at[NBUF * g + NBUF + b]],
                                 rows[b], sem_g[b])
                pltpu.async_copy(dstp_hbm.at[sid, off + NBUF * g + NBUF + b],
                                 dst_v[b], sem_i[b])

        return carry

    lax.fori_loop(0, NG1, body, 0)

    @pl.when(cid == 0)
    def _():
        lax.fori_loop(NG1, NG0, body, 0)

    plsc.subcore_barrier()

    @pl.when(cid == 0)
    def _():
        pltpu.sync_copy(acc_sh.at[pl.ds(sid * RPS, RPS)],
                        p0_hbm.at[pl.ds(sid * RPS, RPS)])

    @pl.when(cid == 1)
    def _():
        pltpu.sync_copy(acc_sh.at[pl.ds(sid * RPS, RPS)],
                        p1_hbm.at[pl.ds(sid * RPS, RPS)])


BR = 640


def _tc1_body(x_ref, w_ref, d0_ref, d1_ref, hs_ref, dinv_ref):
    cnt = d0_ref[:, 0:1] + d1_ref[:, 0:1] + 1.0
    dinv = lax.rsqrt(cnt)
    h = jnp.dot(x_ref[...], w_ref[...], preferred_element_type=jnp.float32)
    hs_ref[...] = dinv * h
    dinv_ref[...] = jnp.broadcast_to(dinv, (BR, DW))


def _tc2_body(p0_ref, p1_ref, hs_ref, dinv_ref, w_ref, b_ref, out_ref):
    dinv = dinv_ref[:, 0:1]
    agg = p0_ref[...] + p1_ref[...] - hs_ref[...]
    a = jnp.maximum(dinv * agg + b_ref[...], 0.0)
    out_ref[...] = dinv * jnp.dot(a, w_ref[...],
                                  preferred_element_type=jnp.float32)


def _tc3_body(p0_ref, p1_ref, hs_ref, dinv_ref, b2_ref,
              wf1_ref, bf1_ref, wf2_ref, bf2_ref, out_ref):
    dinv = dinv_ref[:, 0:1]
    agg = p0_ref[...] + p1_ref[...] - hs_ref[...]
    a = jnp.maximum(dinv * agg + b2_ref[...], 0.0)
    f = jnp.maximum(jnp.dot(a, wf1_ref[...],
                            preferred_element_type=jnp.float32) + bf1_ref[...],
                    0.0)
    logits = jnp.dot(f, wf2_ref[...],
                     preferred_element_type=jnp.float32) + bf2_ref[...]
    m = jnp.max(logits, axis=1, keepdims=True)
    e = jnp.exp(logits - m)
    out_ref[...] = e / jnp.sum(e, axis=1, keepdims=True)


def _row_spec(width):
    return pl.BlockSpec((BR, width), lambda i: (i, 0))


def _full_spec(shape):
    return pl.BlockSpec(shape, lambda i: (0,) * len(shape))


def _tc1(x, w1, deg0, deg1):
    return pl.pallas_call(
        _tc1_body,
        grid=(NP // BR,),
        in_specs=[_row_spec(D), _full_spec((D, D)), _row_spec(DW),
                  _row_spec(DW)],
        out_specs=[_row_spec(D), _row_spec(DW)],
        out_shape=[jax.ShapeDtypeStruct((NP, D), jnp.float32),
                   jax.ShapeDtypeStruct((NP, DW), jnp.float32)],
    )(x, w1, deg0, deg1)


def _tc2(p0, p1, hs, dinv, w2, b1):
    return pl.pallas_call(
        _tc2_body,
        grid=(NP // BR,),
        in_specs=[_row_spec(D), _row_spec(D), _row_spec(D), _row_spec(DW),
                  _full_spec((D, D)), _full_spec((1, D))],
        out_specs=_row_spec(D),
        out_shape=jax.ShapeDtypeStruct((NP, D), jnp.float32),
    )(p0, p1, hs, dinv, w2, b1)


def _tc3(p0, p1, hs, dinv, b2, wf1, bf1, wf2, bf2):
    return pl.pallas_call(
        _tc3_body,
        grid=(NP // BR,),
        in_specs=[_row_spec(D), _row_spec(D), _row_spec(D), _row_spec(DW),
                  _full_spec((1, D)), _full_spec((D, D)), _full_spec((1, D)),
                  _full_spec((D, D)), _full_spec((1, D))],
        out_specs=_row_spec(D),
        out_shape=jax.ShapeDtypeStruct((N, D), jnp.float32),
    )(p0, p1, hs, dinv, b2, wf1, bf1, wf2, bf2)


@jax.jit
def kernel(x, edge_index, W1, b1, W2, b2, Wf1, bf1, Wf2, bf2):
    src = edge_index[0]
    dst = edge_index[1]
    npad = EPAD - E
    srcp = jnp.concatenate(
        [src, jnp.zeros((npad,), jnp.int32)]).reshape(NS, NCHS, CHUNK)
    srcp = jnp.stack(
        [srcp[:, :NCH0],
         jnp.pad(srcp[:, NCH0:], ((0, 0), (0, NCH0 - NCH1), (0, 0)))],
        axis=1).reshape(NW, NCH0, CHUNK)
    dstp = jnp.concatenate(
        [dst, jnp.full((npad,), PAD_DST, jnp.int32)]).reshape(NS, NCHS, CHUNK)

    deg0, deg1 = _sc_degree(dstp.reshape(NW, NCH, CHUNK))

    hs1, dinv = _tc1(x, W1, deg0, deg1)
    p0, p1 = _sc_aggregate(hs1, srcp, dstp)
    hs2 = _tc2(p0, p1, hs1, dinv, W2, b1.reshape(1, D))
    q0, q1 = _sc_aggregate(hs2, srcp, dstp)
    out = _tc3(q0, q1, hs2, dinv, b2.reshape(1, D),
               Wf1, bf1.reshape(1, D), Wf2, bf2.reshape(1, D))
    return out
